# baseline (speedup 1.0000x reference)
import functools

import jax
import jax.numpy as jnp
from jax import lax
from jax.experimental import pallas as pl
from jax.experimental.pallas import tpu as pltpu
from jax.experimental.pallas import tpu_sc as plsc

_N = 4194304
_NC = 2
_NS = 16
_L = 16
_NW = _NC * _NS
_PER_W = _N // _NW
_CHUNK = 16384
_NCHUNK = _PER_W // _CHUNK

_mesh = plsc.VectorSubcoreMesh(core_axis_name="c", subcore_axis_name="s")


@functools.partial(
    pl.kernel,
    out_type=jax.ShapeDtypeStruct((_NW, 3, _L), jnp.float32),
    mesh=_mesh,
    scratch_types=[
        pltpu.VMEM((2, _CHUNK), jnp.float32),
        pltpu.VMEM((2, _CHUNK), jnp.int32),
        pltpu.VMEM((3, _L), jnp.float32),
        pltpu.SemaphoreType.DMA,
        pltpu.SemaphoreType.DMA,
        pltpu.SemaphoreType.DMA,
        pltpu.SemaphoreType.DMA,
    ],
)
def _partial_sums(y_hbm, s_hbm, out_hbm, ybuf, sbuf, accv,
                  sem_y0, sem_y1, sem_s0, sem_s1):
    wid = lax.axis_index("s") * _NC + lax.axis_index("c")
    base = wid * _PER_W
    sems_y = (sem_y0, sem_y1)
    sems_s = (sem_s0, sem_s1)

    def start(k, slot):
        off = base + k * _CHUNK
        cy = pltpu.make_async_copy(
            y_hbm.at[pl.ds(off, _CHUNK)], ybuf.at[slot], sems_y[slot])
        cs = pltpu.make_async_copy(
            s_hbm.at[pl.ds(off, _CHUNK)], sbuf.at[slot], sems_s[slot])
        cy.start()
        cs.start()
        return cy, cs

    def chunk_reduce(acc, slot):
        yb = ybuf.at[slot]
        sb = sbuf.at[slot]

        def body(i, acc):
            ay, ays, asf = acc
            yv = yb[pl.ds(i * _L, _L)]
            sv = sb[pl.ds(i * _L, _L)]
            ay = ay + yv
            ays = ays + jnp.where(sv == 1, yv, jnp.float32(0.0))
            asf = asf + sv.astype(jnp.float32)
            return (ay, ays, asf)

        return lax.fori_loop(0, _CHUNK // _L, body, acc, unroll=8)

    zero = jnp.zeros((_L,), jnp.float32)
    acc = (zero, zero, zero)
    pending = start(0, 0)
    for k in range(_NCHUNK):
        slot = k % 2
        cy, cs = pending
        cy.wait()
        cs.wait()
        if k + 1 < _NCHUNK:
            pending = start(k + 1, (k + 1) % 2)
        acc = chunk_reduce(acc, slot)

    accv[0] = acc[0]
    accv[1] = acc[1]
    accv[2] = acc[2]
    pltpu.sync_copy(accv, out_hbm.at[wid])


def kernel(y_pred, s):
    parts = _partial_sums(y_pred.reshape(-1), s.reshape(-1))
    sums = jnp.sum(parts, axis=(0, 2))
    sum_y = sums[0]
    sum_ys = sums[1]
    cnt1 = sums[2]
    cnt0 = jnp.float32(_N) - cnt1
    mean1 = sum_ys / cnt1
    mean0 = (sum_y - sum_ys) / cnt0
    return jnp.abs(mean0 - mean1)
